# Initial kernel scaffold; baseline (speedup 1.0000x reference)
import jax
import jax.numpy as jnp
from jax.experimental import pallas as pl


def kernel(x, edgeIndex, edgeAttribute, W, b, Wl, bl):
    raise NotImplementedError("write your pallas kernel here")


# speedup vs baseline: 11.1536x; 11.1536x over previous
import functools

import jax
import jax.numpy as jnp
from jax import lax
from jax.experimental import pallas as pl
from jax.experimental.pallas import tpu as pltpu
from jax.experimental.pallas import tpu_sc as plsc

NC = 2
NS = 16
L = 16
CHUNK = 128
_SC_PARAMS = pltpu.CompilerParams(use_tc_tiling_on_sc=False,
                                  needs_layout_passes=False)


def _rsqrt_newton(xv):
    i = plsc.bitcast(xv, jnp.int32)
    i = jnp.int32(0x5F3759DF) - (i >> 1)
    y = plsc.bitcast(i, jnp.float32)
    for _ in range(3):
        y = y * (1.5 - 0.5 * xv * y * y)
    return y


def _sc_deg_dinv(dstp2, wp2, n_pad):
    tot_chunks = dstp2.shape[0]
    chunks = tot_chunks // NS
    rows_pt = n_pad // NS
    mesh = plsc.VectorSubcoreMesh(core_axis_name="c", subcore_axis_name="s")

    @functools.partial(
        pl.kernel,
        out_type=jax.ShapeDtypeStruct((n_pad,), jnp.float32),
        mesh=mesh,
        scratch_types=[
            pltpu.VMEM((chunks, CHUNK), jnp.int32),
            pltpu.VMEM((chunks, CHUNK), jnp.float32),
            pltpu.VMEM((rows_pt,), jnp.float32),
            pltpu.VMEM_SHARED((n_pad,), jnp.float32),
        ],
        compiler_params=_SC_PARAMS,
    )
    def k(dst_hbm, w_hbm, dinv_hbm, dst_v, w_v, deg_v, deg_sh):
        cid = lax.axis_index("c")
        sid = lax.axis_index("s")
        active = cid == 0

        @pl.when(active)
        def _():
            pltpu.sync_copy(dst_hbm.at[pl.ds(sid * chunks, chunks)], dst_v)
            pltpu.sync_copy(w_hbm.at[pl.ds(sid * chunks, chunks)], w_v)

            @pl.loop(0, rows_pt, step=L)
            def _(i):
                deg_v[pl.ds(i, L)] = jnp.zeros((L,), jnp.float32)

            pltpu.sync_copy(deg_v, deg_sh.at[pl.ds(sid * rows_pt, rows_pt)])

        plsc.subcore_barrier()

        @pl.when(active)
        def _():
            @pl.loop(0, chunks)
            def _(j):
                pltpu.sync_copy(w_v.at[j], deg_sh.at[dst_v.at[j]], add=True)

        plsc.subcore_barrier()

        @pl.when(active)
        def _():
            pltpu.sync_copy(deg_sh.at[pl.ds(sid * rows_pt, rows_pt)], deg_v)

            @pl.loop(0, rows_pt, step=L)
            def _(i):
                deg_v[pl.ds(i, L)] = _rsqrt_newton(deg_v[pl.ds(i, L)])

            pltpu.sync_copy(deg_v, dinv_hbm.at[pl.ds(sid * rows_pt, rows_pt)])

    return k(dstp2, wp2)


def _sc_edge_pass(srcp2, dstp2, wp2, dinv, h0, h1, n_pad):
    tot_chunks = srcp2.shape[0]
    dh = h0.shape[1]
    chunks = tot_chunks // NS
    rows_pt = n_pad // NS
    zcopies = rows_pt // CHUNK
    mesh = plsc.VectorSubcoreMesh(core_axis_name="c", subcore_axis_name="s")

    @functools.partial(
        pl.kernel,
        out_type=(jax.ShapeDtypeStruct((n_pad, dh), jnp.float32),
                  jax.ShapeDtypeStruct((n_pad, dh), jnp.float32)),
        mesh=mesh,
        scratch_types=[
            pltpu.VMEM((chunks, CHUNK), jnp.int32),
            pltpu.VMEM((chunks, CHUNK), jnp.int32),
            pltpu.VMEM((chunks, CHUNK), jnp.float32),
            pltpu.VMEM((n_pad,), jnp.float32),
            pltpu.VMEM((CHUNK, dh), jnp.float32),
            pltpu.VMEM_SHARED((n_pad, dh), jnp.float32),
        ],
        compiler_params=_SC_PARAMS,
    )
    def k(src_hbm, dst_hbm, w_hbm, dinv_hbm, h0_hbm, h1_hbm,
          acc0_hbm, acc1_hbm,
          src_v, dst_v, w_v, dinv_v, rows_v, acc_sh):
        cid = lax.axis_index("c")
        sid = lax.axis_index("s")
        base = sid * chunks

        pltpu.sync_copy(src_hbm.at[pl.ds(base, chunks)], src_v)
        pltpu.sync_copy(dst_hbm.at[pl.ds(base, chunks)], dst_v)
        pltpu.sync_copy(w_hbm.at[pl.ds(base, chunks)], w_v)
        pltpu.sync_copy(dinv_hbm, dinv_v)

        @pl.loop(0, chunks)
        def _(j):
            for g in range(CHUNK // L):
                sl = pl.ds(g * L, L)
                si = src_v[j, sl]
                di = dst_v[j, sl]
                c = (w_v[j, sl]
                     * plsc.load_gather(dinv_v, [si])
                     * plsc.load_gather(dinv_v, [di]))
                w_v[j, sl] = c

        @pl.loop(0, CHUNK)
        def _(e):
            for g in range(dh // L):
                rows_v[e, pl.ds(g * L, L)] = jnp.zeros((L,), jnp.float32)

        for r in range(zcopies):
            pltpu.sync_copy(
                rows_v, acc_sh.at[pl.ds(sid * rows_pt + r * CHUNK, CHUNK)])

        plsc.subcore_barrier()

        def main_loop(h_hbm):
            @pl.loop(0, chunks)
            def _(j):
                pltpu.sync_copy(h_hbm.at[src_v.at[j]], rows_v)

                @pl.loop(0, CHUNK // L)
                def _(q):
                    cg = w_v[j, pl.ds(q * L, L)]
                    for t in range(L):
                        cvec = jnp.full((L,), cg[t], jnp.float32)
                        for g in range(dh // L):
                            sl = pl.ds(g * L, L)
                            rows_v[q * L + t, sl] = (
                                rows_v[q * L + t, sl] * cvec)

                pltpu.sync_copy(rows_v, acc_sh.at[dst_v.at[j]], add=True)

        @pl.when(cid == 0)
        def _():
            main_loop(h0_hbm)

        @pl.when(cid == 1)
        def _():
            main_loop(h1_hbm)

        plsc.subcore_barrier()

        rows_sl = pl.ds(sid * rows_pt, rows_pt)

        @pl.when(cid == 0)
        def _():
            pltpu.sync_copy(acc_sh.at[rows_sl], acc0_hbm.at[rows_sl])

        @pl.when(cid == 1)
        def _():
            pltpu.sync_copy(acc_sh.at[rows_sl], acc1_hbm.at[rows_sl])

    return k(srcp2, dstp2, wp2, dinv, h0, h1)


def _tc_pre_body(x_ref, w_ref, wl_ref, bl_ref, h0_ref, h1_ref, xt_ref):
    x = x_ref[...]
    dh = h0_ref.shape[1]
    h = jnp.dot(x, w_ref[...], preferred_element_type=jnp.float32)
    h0_ref[...] = h[:, :dh]
    h1_ref[...] = h[:, dh:]
    xt_ref[...] = (jnp.dot(x, wl_ref[...], preferred_element_type=jnp.float32)
                   + bl_ref[...])


def _tc_post_body(a0_ref, a1_ref, b_ref, xt_ref, o_ref):
    agg = jnp.concatenate([a0_ref[...], a1_ref[...]], axis=1) + b_ref[...]
    o_ref[...] = jnp.where(agg >= 0, agg, 0.01 * agg) + xt_ref[...]


def kernel(x, edgeIndex, edgeAttribute, W, b, Wl, bl):
    n, d = x.shape
    e = edgeIndex.shape[1]
    n_pad = ((n + NS * L - 1) // (NS * L)) * (NS * L)
    grp = NS * CHUNK
    ep = ((e + n + grp - 1) // grp) * grp

    src = edgeIndex[0]
    dst = edgeIndex[1]
    loop = jnp.arange(n, dtype=jnp.int32)
    padz = jnp.zeros((ep - e - n,), jnp.int32)
    srcp2 = jnp.concatenate([src, loop, padz]).reshape(ep // CHUNK, CHUNK)
    dstp2 = jnp.concatenate([dst, loop, padz]).reshape(ep // CHUNK, CHUNK)
    wp2 = jnp.concatenate(
        [edgeAttribute, jnp.ones((n,), jnp.float32),
         jnp.zeros((ep - e - n,), jnp.float32)]).reshape(ep // CHUNK, CHUNK)

    dinv = _sc_deg_dinv(dstp2, wp2, n_pad)

    h0, h1, xt = pl.pallas_call(
        _tc_pre_body,
        out_shape=(jax.ShapeDtypeStruct((n, d // 2), jnp.float32),
                   jax.ShapeDtypeStruct((n, d // 2), jnp.float32),
                   jax.ShapeDtypeStruct((n, d), jnp.float32)),
    )(x, W, Wl, bl.reshape(1, d))

    acc0, acc1 = _sc_edge_pass(srcp2, dstp2, wp2, dinv, h0, h1, n_pad)

    out = pl.pallas_call(
        _tc_post_body,
        out_shape=jax.ShapeDtypeStruct((n, d), jnp.float32),
    )(acc0[:n], acc1[:n], b.reshape(1, d), xt)
    return out


# speedup vs baseline: 13.3702x; 1.1987x over previous
import functools

import jax
import jax.numpy as jnp
from jax import lax
from jax.experimental import pallas as pl
from jax.experimental.pallas import tpu as pltpu
from jax.experimental.pallas import tpu_sc as plsc

NC = 2
NS = 16
L = 16
CHUNK = 128
NBUF = 4
NSECT = 2
_SC_PARAMS = pltpu.CompilerParams(use_tc_tiling_on_sc=False,
                                  needs_layout_passes=False)


def _rsqrt_newton(xv):
    i = plsc.bitcast(xv, jnp.int32)
    i = jnp.int32(0x5F3759DF) - (i >> 1)
    y = plsc.bitcast(i, jnp.float32)
    for _ in range(3):
        y = y * (1.5 - 0.5 * xv * y * y)
    return y


def _sc_deg_dinv(dstp2, wp2, n_pad):
    tot_chunks = dstp2.shape[0]
    chunks = tot_chunks // NS
    rows_pt = n_pad // NS
    mesh = plsc.VectorSubcoreMesh(core_axis_name="c", subcore_axis_name="s")

    @functools.partial(
        pl.kernel,
        out_type=jax.ShapeDtypeStruct((n_pad,), jnp.float32),
        mesh=mesh,
        scratch_types=[
            pltpu.VMEM((chunks, CHUNK), jnp.int32),
            pltpu.VMEM((chunks, CHUNK), jnp.float32),
            pltpu.VMEM((rows_pt,), jnp.float32),
            pltpu.VMEM_SHARED((n_pad,), jnp.float32),
        ],
        compiler_params=_SC_PARAMS,
    )
    def k(dst_hbm, w_hbm, dinv_hbm, dst_v, w_v, deg_v, deg_sh):
        cid = lax.axis_index("c")
        sid = lax.axis_index("s")
        active = cid == 0

        @pl.when(active)
        def _():
            pltpu.sync_copy(dst_hbm.at[pl.ds(sid * chunks, chunks)], dst_v)
            pltpu.sync_copy(w_hbm.at[pl.ds(sid * chunks, chunks)], w_v)

            @pl.loop(0, rows_pt, step=L)
            def _(i):
                deg_v[pl.ds(i, L)] = jnp.zeros((L,), jnp.float32)

            pltpu.sync_copy(deg_v, deg_sh.at[pl.ds(sid * rows_pt, rows_pt)])

        plsc.subcore_barrier()

        @pl.when(active)
        def _():
            @pl.loop(0, chunks)
            def _(j):
                pltpu.sync_copy(w_v.at[j], deg_sh.at[dst_v.at[j]], add=True)

        plsc.subcore_barrier()

        @pl.when(active)
        def _():
            pltpu.sync_copy(deg_sh.at[pl.ds(sid * rows_pt, rows_pt)], deg_v)

            @pl.loop(0, rows_pt, step=L)
            def _(i):
                deg_v[pl.ds(i, L)] = _rsqrt_newton(deg_v[pl.ds(i, L)])

            pltpu.sync_copy(deg_v, dinv_hbm.at[pl.ds(sid * rows_pt, rows_pt)])

    return k(dstp2, wp2)


def _sc_edge_pass(srcp2, dstp2, wp2, dinv, h0, h1, n_pad):
    tot_chunks = srcp2.shape[0]
    dh = h0.shape[1]
    chunks = tot_chunks // NS
    assert chunks % (NBUF * NSECT) == 0
    sect = chunks // NSECT
    rows_pt = n_pad // NS
    zcopies = rows_pt // CHUNK
    mesh = plsc.VectorSubcoreMesh(core_axis_name="c", subcore_axis_name="s")

    @functools.partial(
        pl.kernel,
        out_type=(jax.ShapeDtypeStruct((n_pad, dh), jnp.float32),
                  jax.ShapeDtypeStruct((n_pad, dh), jnp.float32)),
        mesh=mesh,
        scratch_types=[
            pltpu.VMEM((sect, CHUNK), jnp.int32),
            pltpu.VMEM((sect, CHUNK), jnp.int32),
            pltpu.VMEM((sect, CHUNK), jnp.float32),
            pltpu.VMEM((n_pad,), jnp.float32),
            pltpu.VMEM((NBUF, CHUNK, dh), jnp.float32),
            pltpu.VMEM_SHARED((n_pad, dh), jnp.float32),
            pltpu.SemaphoreType.DMA((NBUF,)),
            pltpu.SemaphoreType.DMA((NBUF,)),
        ],
        compiler_params=_SC_PARAMS,
    )
    def k(src_hbm, dst_hbm, w_hbm, dinv_hbm, h0_hbm, h1_hbm,
          acc0_hbm, acc1_hbm,
          src_v, dst_v, w_v, dinv_v, rows_v, acc_sh, semg, sems):
        cid = lax.axis_index("c")
        sid = lax.axis_index("s")

        pltpu.sync_copy(dinv_hbm, dinv_v)

        @pl.loop(0, CHUNK)
        def _(e):
            for g in range(dh // L):
                rows_v[0, e, pl.ds(g * L, L)] = jnp.zeros((L,), jnp.float32)

        for r in range(zcopies):
            pltpu.sync_copy(
                rows_v.at[0],
                acc_sh.at[pl.ds(sid * rows_pt + r * CHUNK, CHUNK)])

        plsc.subcore_barrier()

        def run_section(h_hbm, sbase):
            pltpu.sync_copy(src_hbm.at[pl.ds(sbase, sect)], src_v)
            pltpu.sync_copy(dst_hbm.at[pl.ds(sbase, sect)], dst_v)
            pltpu.sync_copy(w_hbm.at[pl.ds(sbase, sect)], w_v)

            @pl.loop(0, sect)
            def _(j):
                for g in range(CHUNK // L):
                    sl = pl.ds(g * L, L)
                    c = (w_v[j, sl]
                         * plsc.load_gather(dinv_v, [src_v[j, sl]])
                         * plsc.load_gather(dinv_v, [dst_v[j, sl]]))
                    w_v[j, sl] = c

            def gather_start(j, b):
                pltpu.async_copy(
                    h_hbm.at[src_v.at[j]], rows_v.at[b], semg.at[b])

            def gather_wait(j, b):
                pltpu.make_async_copy(
                    h_hbm.at[src_v.at[j]], rows_v.at[b], semg.at[b]).wait()

            def scatter_start(j, b):
                pltpu.async_copy(
                    rows_v.at[b], acc_sh.at[dst_v.at[j]], sems.at[b],
                    add=True)

            def scatter_wait(j, b):
                pltpu.make_async_copy(
                    rows_v.at[b], acc_sh.at[dst_v.at[j]], sems.at[b]).wait()

            gather_start(0, 0)
            gather_start(1, 1)

            @pl.loop(0, sect, step=NBUF)
            def _(j0):
                for bo in range(NBUF):
                    jj = j0 + bo
                    gather_wait(jj, bo)

                    @pl.loop(0, CHUNK // L)
                    def _(q):
                        cg = w_v[jj, pl.ds(q * L, L)]
                        for t in range(L):
                            cvec = jnp.full((L,), cg[t], jnp.float32)
                            for g in range(dh // L):
                                sl = pl.ds(g * L, L)
                                rows_v[bo, q * L + t, sl] = (
                                    rows_v[bo, q * L + t, sl] * cvec)

                    scatter_start(jj, bo)

                    bn = (bo + 2) % NBUF
                    jn = jj + 2

                    @pl.when(jn < sect)
                    def _():
                        @pl.when(jj >= 2)
                        def _():
                            scatter_wait(jj - 2, bn)

                        gather_start(jn, bn)

            for bo in range(NBUF):
                scatter_wait(sect - NBUF + bo, bo)

        def main_loop(h_hbm):
            for t in range(NSECT):
                run_section(h_hbm, sid * chunks + t * sect)

        @pl.when(cid == 0)
        def _():
            main_loop(h0_hbm)

        @pl.when(cid == 1)
        def _():
            main_loop(h1_hbm)

        plsc.subcore_barrier()

        rows_sl = pl.ds(sid * rows_pt, rows_pt)

        @pl.when(cid == 0)
        def _():
            pltpu.sync_copy(acc_sh.at[rows_sl], acc0_hbm.at[rows_sl])

        @pl.when(cid == 1)
        def _():
            pltpu.sync_copy(acc_sh.at[rows_sl], acc1_hbm.at[rows_sl])

    return k(srcp2, dstp2, wp2, dinv, h0, h1)


def _tc_pre_body(x_ref, w_ref, wl_ref, bl_ref, h0_ref, h1_ref, xt_ref):
    x = x_ref[...]
    dh = h0_ref.shape[1]
    h = jnp.dot(x, w_ref[...], preferred_element_type=jnp.float32)
    h0_ref[...] = h[:, :dh]
    h1_ref[...] = h[:, dh:]
    xt_ref[...] = (jnp.dot(x, wl_ref[...], preferred_element_type=jnp.float32)
                   + bl_ref[...])


def _tc_post_body(a0_ref, a1_ref, b_ref, xt_ref, o_ref):
    agg = jnp.concatenate([a0_ref[...], a1_ref[...]], axis=1) + b_ref[...]
    o_ref[...] = jnp.where(agg >= 0, agg, 0.01 * agg) + xt_ref[...]


def kernel(x, edgeIndex, edgeAttribute, W, b, Wl, bl):
    n, d = x.shape
    e = edgeIndex.shape[1]
    n_pad = ((n + NS * L - 1) // (NS * L)) * (NS * L)
    grp = NS * CHUNK * NBUF * NSECT
    ep = ((e + n + grp - 1) // grp) * grp

    src = edgeIndex[0]
    dst = edgeIndex[1]
    loop = jnp.arange(n, dtype=jnp.int32)
    padz = jnp.zeros((ep - e - n,), jnp.int32)
    srcp2 = jnp.concatenate([src, loop, padz]).reshape(ep // CHUNK, CHUNK)
    dstp2 = jnp.concatenate([dst, loop, padz]).reshape(ep // CHUNK, CHUNK)
    wp2 = jnp.concatenate(
        [edgeAttribute, jnp.ones((n,), jnp.float32),
         jnp.zeros((ep - e - n,), jnp.float32)]).reshape(ep // CHUNK, CHUNK)

    dinv = _sc_deg_dinv(dstp2, wp2, n_pad)

    h0, h1, xt = pl.pallas_call(
        _tc_pre_body,
        out_shape=(jax.ShapeDtypeStruct((n, d // 2), jnp.float32),
                   jax.ShapeDtypeStruct((n, d // 2), jnp.float32),
                   jax.ShapeDtypeStruct((n, d), jnp.float32)),
    )(x, W, Wl, bl.reshape(1, d))

    acc0, acc1 = _sc_edge_pass(srcp2, dstp2, wp2, dinv, h0, h1, n_pad)

    out = pl.pallas_call(
        _tc_post_body,
        out_shape=jax.ShapeDtypeStruct((n, d), jnp.float32),
    )(acc0[:n], acc1[:n], b.reshape(1, d), xt)
    return out


# speedup vs baseline: 29.4938x; 2.2059x over previous
import functools

import jax
import jax.numpy as jnp
from jax import lax
from jax.experimental import pallas as pl
from jax.experimental.pallas import tpu as pltpu
from jax.experimental.pallas import tpu_sc as plsc

NC = 2
NS = 16
L = 16
CHUNK = 128
NBUF = 4
NSECT = 2
_SC_PARAMS = pltpu.CompilerParams(use_tc_tiling_on_sc=False,
                                  needs_layout_passes=False)


def _rsqrt_newton(xv):
    i = plsc.bitcast(xv, jnp.int32)
    i = jnp.int32(0x5F3759DF) - (i >> 1)
    y = plsc.bitcast(i, jnp.float32)
    for _ in range(3):
        y = y * (1.5 - 0.5 * xv * y * y)
    return y


def _sc_edge_pass(srcp2, dstp2, wp2, h0, h1, n_pad):
    tot_chunks = srcp2.shape[0]
    dh = h0.shape[1]
    chunks = tot_chunks // NS
    assert chunks % (NBUF * NSECT) == 0
    sect = chunks // NSECT
    rows_pt = n_pad // NS
    zcopies = rows_pt // CHUNK
    mesh = plsc.VectorSubcoreMesh(core_axis_name="c", subcore_axis_name="s")

    @functools.partial(
        pl.kernel,
        out_type=(jax.ShapeDtypeStruct((n_pad, dh), jnp.float32),
                  jax.ShapeDtypeStruct((n_pad, dh), jnp.float32)),
        mesh=mesh,
        scratch_types=[
            pltpu.VMEM((sect, CHUNK), jnp.int32),
            pltpu.VMEM((sect, CHUNK), jnp.int32),
            pltpu.VMEM((sect, CHUNK), jnp.float32),
            pltpu.VMEM((n_pad,), jnp.float32),
            pltpu.VMEM((NBUF, CHUNK, dh), jnp.float32),
            pltpu.VMEM_SHARED((n_pad, dh), jnp.float32),
            pltpu.VMEM_SHARED((n_pad,), jnp.float32),
            pltpu.SemaphoreType.DMA((NBUF,)),
            pltpu.SemaphoreType.DMA((NBUF,)),
        ],
        compiler_params=_SC_PARAMS,
    )
    def k(src_hbm, dst_hbm, w_hbm, h0_hbm, h1_hbm,
          acc0_hbm, acc1_hbm,
          src_v, dst_v, w_v, dinv_v, rows_v, acc_sh, deg_sh, semg, sems):
        cid = lax.axis_index("c")
        sid = lax.axis_index("s")
        my_rows = pl.ds(sid * rows_pt, rows_pt)

        with jax.named_scope("edge_init"):
            @pl.loop(0, rows_pt, step=L)
            def _(i):
                dinv_v[pl.ds(i, L)] = jnp.zeros((L,), jnp.float32)

            pltpu.sync_copy(dinv_v.at[pl.ds(0, rows_pt)], deg_sh.at[my_rows])

            @pl.loop(0, CHUNK)
            def _(e):
                for g in range(dh // L):
                    rows_v[0, e, pl.ds(g * L, L)] = jnp.zeros(
                        (L,), jnp.float32)

            for r in range(zcopies):
                pltpu.sync_copy(
                    rows_v.at[0],
                    acc_sh.at[pl.ds(sid * rows_pt + r * CHUNK, CHUNK)])

        plsc.subcore_barrier()

        with jax.named_scope("edge_deg"):
            for t in range(NSECT):
                sbase = sid * chunks + t * sect
                pltpu.sync_copy(dst_hbm.at[pl.ds(sbase, sect)], dst_v)
                pltpu.sync_copy(w_hbm.at[pl.ds(sbase, sect)], w_v)

                @pl.loop(0, sect)
                def _(j):
                    pltpu.sync_copy(w_v.at[j], deg_sh.at[dst_v.at[j]],
                                    add=True)

        plsc.subcore_barrier()

        with jax.named_scope("edge_dinv"):
            pltpu.sync_copy(deg_sh.at[my_rows], dinv_v.at[pl.ds(0, rows_pt)])

            @pl.loop(0, rows_pt, step=L)
            def _(i):
                dinv_v[pl.ds(i, L)] = _rsqrt_newton(dinv_v[pl.ds(i, L)])

            pltpu.sync_copy(dinv_v.at[pl.ds(0, rows_pt)], deg_sh.at[my_rows])

        plsc.subcore_barrier()

        with jax.named_scope("edge_dinv_bcast"):
            pltpu.sync_copy(deg_sh, dinv_v)

        def run_section(h_hbm, sbase):
            with jax.named_scope("edge_load_c"):
                pltpu.sync_copy(src_hbm.at[pl.ds(sbase, sect)], src_v)
                pltpu.sync_copy(dst_hbm.at[pl.ds(sbase, sect)], dst_v)
                pltpu.sync_copy(w_hbm.at[pl.ds(sbase, sect)], w_v)

                @pl.loop(0, sect)
                def _(j):
                    for g in range(CHUNK // L):
                        sl = pl.ds(g * L, L)
                        c = (w_v[j, sl]
                             * plsc.load_gather(dinv_v, [src_v[j, sl]])
                             * plsc.load_gather(dinv_v, [dst_v[j, sl]]))
                        w_v[j, sl] = c

            def gather_start(j, b):
                pltpu.async_copy(
                    h_hbm.at[src_v.at[j]], rows_v.at[b], semg.at[b])

            def gather_wait(j, b):
                pltpu.make_async_copy(
                    h_hbm.at[src_v.at[j]], rows_v.at[b], semg.at[b]).wait()

            def scatter_start(j, b):
                pltpu.async_copy(
                    rows_v.at[b], acc_sh.at[dst_v.at[j]], sems.at[b],
                    add=True)

            def scatter_wait(j, b):
                pltpu.make_async_copy(
                    rows_v.at[b], acc_sh.at[dst_v.at[j]], sems.at[b]).wait()

            with jax.named_scope("edge_pipe"):
                _pipe_loop(gather_start, gather_wait,
                           scatter_start, scatter_wait)

        def _pipe_loop(gather_start, gather_wait, scatter_start,
                       scatter_wait):
            gather_start(0, 0)
            gather_start(1, 1)

            @pl.loop(0, sect, step=NBUF)
            def _(j0):
                for bo in range(NBUF):
                    jj = j0 + bo
                    gather_wait(jj, bo)

                    @pl.loop(0, CHUNK // L)
                    def _(q):
                        cg = w_v[jj, pl.ds(q * L, L)]
                        for t in range(0, L, 2):
                            e0 = q * L + t
                            e1 = e0 + 1
                            c0 = jnp.full((L,), cg[t], jnp.float32)
                            c1 = jnp.full((L,), cg[t + 1], jnp.float32)
                            ng = dh // L
                            vals = ([rows_v[bo, e0, pl.ds(g * L, L)]
                                     for g in range(ng)]
                                    + [rows_v[bo, e1, pl.ds(g * L, L)]
                                       for g in range(ng)])
                            prods = ([v * c0 for v in vals[:ng]]
                                     + [v * c1 for v in vals[ng:]])
                            for g in range(ng):
                                rows_v[bo, e0, pl.ds(g * L, L)] = prods[g]
                            for g in range(ng):
                                rows_v[bo, e1, pl.ds(g * L, L)] = (
                                    prods[ng + g])

                    scatter_start(jj, bo)

                    bn = (bo + 2) % NBUF
                    jn = jj + 2

                    @pl.when(jn < sect)
                    def _():
                        @pl.when(jj >= 2)
                        def _():
                            scatter_wait(jj - 2, bn)

                        gather_start(jn, bn)

            for bo in range(NBUF):
                scatter_wait(sect - NBUF + bo, bo)

        def main_loop(h_hbm):
            for t in range(NSECT):
                run_section(h_hbm, sid * chunks + t * sect)

        @pl.when(cid == 0)
        def _():
            main_loop(h0_hbm)

        @pl.when(cid == 1)
        def _():
            main_loop(h1_hbm)

        with jax.named_scope("edge_bar2"):
            plsc.subcore_barrier()

        with jax.named_scope("edge_drain"):
            @pl.when(cid == 0)
            def _():
                pltpu.sync_copy(acc_sh.at[my_rows], acc0_hbm.at[my_rows])

            @pl.when(cid == 1)
            def _():
                pltpu.sync_copy(acc_sh.at[my_rows], acc1_hbm.at[my_rows])

    return k(srcp2, dstp2, wp2, h0, h1)


def _tc_pre_body(x_ref, w_ref, wl_ref, bl_ref, h0_ref, h1_ref, xt_ref):
    x = x_ref[...]
    dh = h0_ref.shape[1]
    h = jnp.dot(x, w_ref[...], preferred_element_type=jnp.float32)
    h0_ref[...] = h[:, :dh]
    h1_ref[...] = h[:, dh:]
    xt_ref[...] = (jnp.dot(x, wl_ref[...], preferred_element_type=jnp.float32)
                   + bl_ref[...])


def _tc_post_body(a0_ref, a1_ref, b_ref, xt_ref, o_ref):
    n = o_ref.shape[0]
    agg = jnp.concatenate([a0_ref[...][:n], a1_ref[...][:n]],
                          axis=1) + b_ref[...]
    o_ref[...] = jnp.where(agg >= 0, agg, 0.01 * agg) + xt_ref[...]


def kernel(x, edgeIndex, edgeAttribute, W, b, Wl, bl):
    n, d = x.shape
    e = edgeIndex.shape[1]
    n_pad = ((n + NS * L - 1) // (NS * L)) * (NS * L)
    grp = NS * CHUNK * NBUF * NSECT
    ep = ((e + n + grp - 1) // grp) * grp

    src = edgeIndex[0]
    dst = edgeIndex[1]
    loop = jnp.arange(n, dtype=jnp.int32)
    padz = jnp.arange(ep - e - n, dtype=jnp.int32) % n
    srcp2 = jnp.concatenate([src, loop, padz]).reshape(ep // CHUNK, CHUNK)
    dstp2 = jnp.concatenate([dst, loop, padz]).reshape(ep // CHUNK, CHUNK)
    wp2 = jnp.concatenate(
        [edgeAttribute, jnp.ones((n,), jnp.float32),
         jnp.zeros((ep - e - n,), jnp.float32)]).reshape(ep // CHUNK, CHUNK)

    h0, h1, xt = pl.pallas_call(
        _tc_pre_body,
        out_shape=(jax.ShapeDtypeStruct((n, d // 2), jnp.float32),
                   jax.ShapeDtypeStruct((n, d // 2), jnp.float32),
                   jax.ShapeDtypeStruct((n, d), jnp.float32)),
    )(x, W, Wl, bl.reshape(1, d))

    acc0, acc1 = _sc_edge_pass(srcp2, dstp2, wp2, h0, h1, n_pad)

    out = pl.pallas_call(
        _tc_post_body,
        out_shape=jax.ShapeDtypeStruct((n, d), jnp.float32),
    )(acc0, acc1, b.reshape(1, d), xt)
    return out
